# baseline (speedup 1.0000x reference)
import functools

import jax
import jax.numpy as jnp
from jax import lax
from jax.experimental import pallas as pl
from jax.experimental.pallas import tpu as pltpu
from jax.experimental.pallas import tpu_sc as plsc

N_NODES = 10000
N_EDGES = 320000
D_FEAT = 128

NW = 32
PER_W = N_EDGES // NW
CHUNK = 80
NCHUNK = PER_W // CHUNK

LANES = 16
D_VECS = D_FEAT // LANES


def _sc_kernel_body(h_hbm, src_hbm, dst_hbm, out_hbm,
                    idx_s, idx_d, rows_s, rows_d, out_v, t16, sem0, sem1):
    wid = lax.axis_index("s") * 2 + lax.axis_index("c")
    wbase = wid * PER_W
    lane_iota = lax.iota(jnp.int32, LANES)

    pltpu.sync_copy(src_hbm.at[pl.ds(wbase, PER_W)], idx_s)
    pltpu.sync_copy(dst_hbm.at[pl.ds(wbase, PER_W)], idx_d)

    def gather_copies(c, p, sem):
        cp_s = pltpu.make_async_copy(
            h_hbm.at[idx_s.at[pl.ds(c * CHUNK, CHUNK)]], rows_s.at[p], sem)
        cp_d = pltpu.make_async_copy(
            h_hbm.at[idx_d.at[pl.ds(c * CHUNK, CHUNK)]], rows_d.at[p], sem)
        return cp_s, cp_d

    def start(c, p, sem):
        cp_s, cp_d = gather_copies(c, p, sem)
        cp_s.start()
        cp_d.start()

    def wait(c, p, sem):
        cp_s, cp_d = gather_copies(c, p, sem)
        cp_s.wait()
        cp_d.wait()

    start(0, 0, sem0)
    start(1, 1, sem1)

    def chunk_body(c, carry):
        p = lax.rem(c, 2)

        def do_chunk(p_const, sem_p):
            wait(c, p_const, sem_p)

            shift16 = jnp.full((LANES,), 16, jnp.int32)

            def group_body(g, carry2):
                gbase = g * LANES
                for j in range(LANES):
                    acc = None
                    for k in range(D_FEAT // (2 * LANES)):
                        vs = rows_s[p_const, gbase + j, pl.ds(k * LANES, LANES)]
                        vd = rows_d[p_const, gbase + j, pl.ds(k * LANES, LANES)]
                        s_lo = plsc.bitcast(lax.shift_left(vs, shift16),
                                            jnp.float32)
                        d_lo = plsc.bitcast(lax.shift_left(vd, shift16),
                                            jnp.float32)
                        s_hi = plsc.bitcast(vs, jnp.float32)
                        d_hi = plsc.bitcast(vd, jnp.float32)
                        part = s_lo * d_lo + s_hi * d_hi
                        acc = part if acc is None else acc + part
                    t16[pl.ds(j * (LANES + 1), LANES)] = acc
                row_base = lane_iota * (LANES + 1)
                r = plsc.load_gather(t16, [row_base])
                for i in range(1, LANES):
                    r = r + plsc.load_gather(t16, [row_base + i])
                out_v[pl.ds(c * CHUNK + gbase, LANES)] = r
                return carry2

            for g_static in range(CHUNK // LANES):
                group_body(g_static, 0)

            @pl.when(c + 2 < NCHUNK)
            def _():
                start(c + 2, p_const, sem_p)

        @pl.when(p == 0)
        def _():
            do_chunk(0, sem0)

        @pl.when(p == 1)
        def _():
            do_chunk(1, sem1)

        return carry

    lax.fori_loop(0, NCHUNK, chunk_body, 0)
    pltpu.sync_copy(out_v, out_hbm.at[pl.ds(wbase, PER_W)])


@jax.jit
def _scores(h, src, dst):
    mesh = plsc.VectorSubcoreMesh(core_axis_name="c", subcore_axis_name="s")
    kfn = functools.partial(
        pl.kernel,
        mesh=mesh,
        compiler_params=pltpu.CompilerParams(
            needs_layout_passes=False, use_tc_tiling_on_sc=False),
        out_type=jax.ShapeDtypeStruct((N_EDGES,), jnp.float32),
        scratch_types=[
            pltpu.VMEM((PER_W,), jnp.int32),
            pltpu.VMEM((PER_W,), jnp.int32),
            pltpu.VMEM((2, CHUNK, D_FEAT // 2), jnp.int32),
            pltpu.VMEM((2, CHUNK, D_FEAT // 2), jnp.int32),
            pltpu.VMEM((PER_W,), jnp.float32),
            pltpu.VMEM((LANES * (LANES + 1),), jnp.float32),
            pltpu.SemaphoreType.DMA,
            pltpu.SemaphoreType.DMA,
        ],
    )(_sc_kernel_body)
    return kfn(h, src, dst)


def kernel(h, edge_index):
    src = edge_index[0].astype(jnp.int32)
    dst = edge_index[1].astype(jnp.int32)
    h_packed = lax.bitcast_convert_type(
        h.astype(jnp.bfloat16).reshape(N_NODES, D_FEAT // 2, 2), jnp.int32)
    scores = _scores(h_packed, src, dst)
    return scores.reshape(N_EDGES, 1)


# speedup vs baseline: 1.2825x; 1.2825x over previous
import functools

import jax
import jax.numpy as jnp
from jax import lax
from jax.experimental import pallas as pl
from jax.experimental.pallas import tpu as pltpu
from jax.experimental.pallas import tpu_sc as plsc

N_NODES = 10000
N_EDGES = 320000
D_FEAT = 128

NW = 32
PER_W = N_EDGES // NW
CHUNK = 80
NCHUNK = PER_W // CHUNK

LANES = 16
D_VECS = D_FEAT // LANES


def _sc_kernel_body(h_hbm, src_hbm, dst_hbm, out_hbm,
                    idx_s, idx_d, rows_s, rows_d, out_v, sem0, sem1):
    wid = lax.axis_index("s") * 2 + lax.axis_index("c")
    wbase = wid * PER_W
    lane_iota = lax.iota(jnp.int32, LANES)

    pltpu.sync_copy(src_hbm.at[pl.ds(wbase, PER_W)], idx_s)
    pltpu.sync_copy(dst_hbm.at[pl.ds(wbase, PER_W)], idx_d)

    def gather_copies(c, p, sem):
        cp_s = pltpu.make_async_copy(
            h_hbm.at[idx_s.at[pl.ds(c * CHUNK, CHUNK)]], rows_s.at[p], sem)
        cp_d = pltpu.make_async_copy(
            h_hbm.at[idx_d.at[pl.ds(c * CHUNK, CHUNK)]], rows_d.at[p], sem)
        return cp_s, cp_d

    def start(c, p, sem):
        cp_s, cp_d = gather_copies(c, p, sem)
        cp_s.start()
        cp_d.start()

    def wait(c, p, sem):
        cp_s, cp_d = gather_copies(c, p, sem)
        cp_s.wait()
        cp_d.wait()

    start(0, 0, sem0)
    start(1, 1, sem1)

    def chunk_body(c, carry):
        p = lax.rem(c, 2)

        def do_chunk(p_const, sem_p):
            wait(c, p_const, sem_p)

            shift16 = jnp.full((LANES,), 16, jnp.int32)
            rot_idx = {d: lane_iota ^ d for d in (8, 4, 2, 1)}
            sel_msk = {d: (lane_iota & d) == 0 for d in (8, 4, 2, 1)}

            def xmerge(x, y, d):
                xr = jnp.take_along_axis(x, rot_idx[d], axis=0,
                                         mode="promise_in_bounds")
                yr = jnp.take_along_axis(y, rot_idx[d], axis=0,
                                         mode="promise_in_bounds")
                m = sel_msk[d]
                return jnp.where(m, x, yr) + jnp.where(m, xr, y)

            def group_body(g, carry2):
                gbase = g * LANES
                accs = []
                for j in range(LANES):
                    acc = None
                    for k in range(D_FEAT // (2 * LANES)):
                        vs = rows_s[p_const, gbase + j, pl.ds(k * LANES, LANES)]
                        vd = rows_d[p_const, gbase + j, pl.ds(k * LANES, LANES)]
                        s_lo = plsc.bitcast(lax.shift_left(vs, shift16),
                                            jnp.float32)
                        d_lo = plsc.bitcast(lax.shift_left(vd, shift16),
                                            jnp.float32)
                        s_hi = plsc.bitcast(vs, jnp.float32)
                        d_hi = plsc.bitcast(vd, jnp.float32)
                        part = s_lo * d_lo + s_hi * d_hi
                        acc = part if acc is None else acc + part
                    accs.append(acc)
                for d in (8, 4, 2, 1):
                    accs = [xmerge(accs[i], accs[i + d], d)
                            for i in range(len(accs)) if not (i & d)]
                out_v[pl.ds(c * CHUNK + gbase, LANES)] = accs[0]
                return carry2

            lax.fori_loop(0, CHUNK // LANES, group_body, 0)

            @pl.when(c + 2 < NCHUNK)
            def _():
                start(c + 2, p_const, sem_p)

        @pl.when(p == 0)
        def _():
            do_chunk(0, sem0)

        @pl.when(p == 1)
        def _():
            do_chunk(1, sem1)

        return carry

    lax.fori_loop(0, NCHUNK, chunk_body, 0)
    pltpu.sync_copy(out_v, out_hbm.at[pl.ds(wbase, PER_W)])


@jax.jit
def _scores(h, src, dst):
    mesh = plsc.VectorSubcoreMesh(core_axis_name="c", subcore_axis_name="s")
    kfn = functools.partial(
        pl.kernel,
        mesh=mesh,
        compiler_params=pltpu.CompilerParams(
            needs_layout_passes=False, use_tc_tiling_on_sc=False),
        out_type=jax.ShapeDtypeStruct((N_EDGES,), jnp.float32),
        scratch_types=[
            pltpu.VMEM((PER_W,), jnp.int32),
            pltpu.VMEM((PER_W,), jnp.int32),
            pltpu.VMEM((2, CHUNK, D_FEAT // 2), jnp.int32),
            pltpu.VMEM((2, CHUNK, D_FEAT // 2), jnp.int32),
            pltpu.VMEM((PER_W,), jnp.float32),
            pltpu.SemaphoreType.DMA,
            pltpu.SemaphoreType.DMA,
        ],
    )(_sc_kernel_body)
    return kfn(h, src, dst)


def kernel(h, edge_index):
    src = edge_index[0].astype(jnp.int32)
    dst = edge_index[1].astype(jnp.int32)
    h_packed = lax.bitcast_convert_type(
        h.astype(jnp.bfloat16).reshape(N_NODES, D_FEAT // 2, 2), jnp.int32)
    scores = _scores(h_packed, src, dst)
    return scores.reshape(N_EDGES, 1)
